# baseline (speedup 1.0000x reference)
import jax
import jax.numpy as jnp
from jax.experimental import pallas as pl

_NUM_CLASSES = 10
_NUM_QUERY = 900
_NUM_PROPAGATED = 256
_G = 5
_BBOX_NOISE_SCALE = 0.4
_NOISE_THRESH = 0.5


def _body(gt_ref, lab_ref, rp_ref, rnd_ref, cor_ref,
          padded_ref, labels_ref, bboxes_ref):
    B, N = lab_ref.shape
    G = _G
    pad = G * N

    gt = gt_ref[...]
    lo_k = (-65.0, -65.0, -8.0)
    rng_k = (130.0, 130.0, 16.0)
    lab = lab_ref[...]
    for g in range(G):
        cols = []
        for k in range(3):
            c_k = gt[:, :, k]
            d_k = gt[:, :, 3 + k] * 0.5
            r_gk = rnd_ref[pl.ds((k * G + g) * B, B), :]
            noisy = c_k + r_gk * d_k * _BBOX_NOISE_SCALE
            noisy = jnp.clip((noisy - lo_k[k]) / rng_k[k], 0.0, 1.0)
            cols.append(noisy)
        padded_ref[:, g * N:(g + 1) * N, :] = jnp.stack(cols, axis=-1)
        labels_ref[g] = jnp.where(cor_ref[g] != 0, _NUM_CLASSES, lab)
        bboxes_ref[g] = gt
    padded_ref[:, pad:, :] = jnp.broadcast_to(rp_ref[...][None],
                                              (B, _NUM_QUERY, 3))


def kernel(batch_size, reference_points, gt_bboxes_3d, gt_labels):
    B, N = gt_labels.shape
    G = _G
    pad = G * N
    total_q = pad + _NUM_QUERY
    tgt = total_q + _NUM_PROPAGATED

    with jax.ensure_compile_time_eval():
        nk = jax.random.key(12345)
        rand_prob = jax.random.uniform(nk, (G * B * N, 3), dtype=jnp.float32)
        rand_prob = rand_prob * 2.0 - 1.0
        corrupt = (jnp.linalg.norm(rand_prob, axis=1) > _NOISE_THRESH)
        rnd = jnp.transpose(rand_prob.reshape(G, B, N, 3),
                            (3, 0, 1, 2)).reshape(3 * G * B, N)
        cor = corrupt.astype(jnp.int32).reshape(G, B, N)

    out_shape = (
        jax.ShapeDtypeStruct((B, total_q, 3), jnp.float32),
        jax.ShapeDtypeStruct((G, B, N), jnp.int32),
        jax.ShapeDtypeStruct((G, B, N, 9), jnp.float32),
    )
    padded, labels, bboxes = pl.pallas_call(
        _body,
        out_shape=out_shape,
    )(gt_bboxes_3d, gt_labels, reference_points, rnd, cor)
    row = jax.lax.broadcasted_iota(jnp.int32, (tgt, tgt), 0)
    col = jax.lax.broadcasted_iota(jnp.int32, (tgt, tgt), 1)
    mask = (col < pad) & ((row // N) != (col // N))
    return (padded, mask,
            labels.reshape(G * B * N), bboxes.reshape(G * B * N, 9))


# speedup vs baseline: 1.1625x; 1.1625x over previous
import jax
import jax.numpy as jnp
from jax.experimental import pallas as pl

_NUM_CLASSES = 10
_NUM_QUERY = 900
_NUM_PROPAGATED = 256
_G = 5
_BBOX_NOISE_SCALE = 0.4
_NOISE_THRESH = 0.5
_PC_LO = (-65.0, -65.0, -8.0)
_PC_RNG = (130.0, 130.0, 16.0)


def _body(gt_ref, lab_ref, rp_ref, rnd_ref, cor_ref,
          padded_t_ref, labels_ref, bboxes_ref):
    B, N = lab_ref.shape
    G = _G
    pad = G * N

    gt = gt_ref[...]
    rp = rp_ref[...]
    for k in range(3):
        c_k = gt[:, :, k]
        d2_k = gt[:, :, 3 + k] * 0.5
        for g in range(G):
            r = rnd_ref[pl.ds((k * G + g) * B, B), :]
            noisy = c_k + r * d2_k * _BBOX_NOISE_SCALE
            noisy = jnp.clip((noisy - _PC_LO[k]) / _PC_RNG[k], 0.0, 1.0)
            padded_t_ref[k, :, g * N:(g + 1) * N] = noisy
        padded_t_ref[k, :, pad:] = jnp.broadcast_to(rp[:, k][None, :],
                                                    (B, _NUM_QUERY))

    lab = lab_ref[...]
    for g in range(G):
        labels_ref[g] = jnp.where(cor_ref[g] != 0, _NUM_CLASSES, lab)
        bboxes_ref[g] = gt


def kernel(batch_size, reference_points, gt_bboxes_3d, gt_labels):
    B, N = gt_labels.shape
    G = _G
    pad = G * N
    total_q = pad + _NUM_QUERY
    tgt = total_q + _NUM_PROPAGATED

    with jax.ensure_compile_time_eval():
        nk = jax.random.key(12345)
        rand_prob = jax.random.uniform(nk, (G * B * N, 3), dtype=jnp.float32)
        rand_prob = rand_prob * 2.0 - 1.0
        corrupt = (jnp.linalg.norm(rand_prob, axis=1) > _NOISE_THRESH)
        rnd = jnp.transpose(rand_prob.reshape(G, B, N, 3),
                            (3, 0, 1, 2)).reshape(3 * G * B, N)
        cor = corrupt.astype(jnp.int32).reshape(G, B, N)

    out_shape = (
        jax.ShapeDtypeStruct((3, B, total_q), jnp.float32),
        jax.ShapeDtypeStruct((G, B, N), jnp.int32),
        jax.ShapeDtypeStruct((G, B, N, 9), jnp.float32),
    )
    padded_t, labels, bboxes = pl.pallas_call(
        _body,
        out_shape=out_shape,
    )(gt_bboxes_3d, gt_labels, reference_points, rnd, cor)
    padded = jnp.transpose(padded_t, (1, 2, 0))
    row = jax.lax.broadcasted_iota(jnp.int32, (tgt, tgt), 0)
    col = jax.lax.broadcasted_iota(jnp.int32, (tgt, tgt), 1)
    mask = (col < pad) & ((row // N) != (col // N))
    return (padded, mask,
            labels.reshape(G * B * N), bboxes.reshape(G * B * N, 9))


# speedup vs baseline: 1.2040x; 1.0357x over previous
import jax
import jax.numpy as jnp
from jax.experimental import pallas as pl

_NUM_CLASSES = 10
_NUM_QUERY = 900
_NUM_PROPAGATED = 256
_G = 5
_BBOX_NOISE_SCALE = 0.4
_NOISE_THRESH = 0.5
_PC_LO = (-65.0, -65.0, -8.0)
_PC_RNG = (130.0, 130.0, 16.0)


def _body(gt_ref, lab_ref, rp_ref, rnd_ref, cor_ref,
          padded_t_ref, labels_ref, bboxes_ref):
    B, N = lab_ref.shape
    G = _G
    pad = G * N

    gt = gt_ref[...]
    rp = rp_ref[...]
    for k in range(3):
        c_k = gt[:, :, k]
        d2_k = gt[:, :, 3 + k] * 0.5
        for g in range(G):
            r = rnd_ref[pl.ds((k * G + g) * B, B), :]
            noisy = c_k + r * d2_k * _BBOX_NOISE_SCALE
            noisy = jnp.clip((noisy - _PC_LO[k]) / _PC_RNG[k], 0.0, 1.0)
            padded_t_ref[k, :, g * N:(g + 1) * N] = noisy
        padded_t_ref[k, :, pad:] = jnp.broadcast_to(rp[:, k][None, :],
                                                    (B, _NUM_QUERY))

    lab = lab_ref[...]
    for g in range(G):
        labels_ref[g] = jnp.where(cor_ref[g] != 0, _NUM_CLASSES, lab)
        bboxes_ref[g] = gt


def kernel(batch_size, reference_points, gt_bboxes_3d, gt_labels):
    B, N = gt_labels.shape
    G = _G
    pad = G * N
    total_q = pad + _NUM_QUERY
    tgt = total_q + _NUM_PROPAGATED

    with jax.ensure_compile_time_eval():
        nk = jax.random.key(12345)
        rand_prob = jax.random.uniform(nk, (G * B * N, 3), dtype=jnp.float32)
        rand_prob = rand_prob * 2.0 - 1.0
        corrupt = (jnp.linalg.norm(rand_prob, axis=1) > _NOISE_THRESH)
        rnd = jnp.transpose(rand_prob.reshape(G, B, N, 3),
                            (3, 0, 1, 2)).reshape(3 * G * B, N)
        cor = corrupt.astype(jnp.int32).reshape(G, B, N)

    out_shape = (
        jax.ShapeDtypeStruct((3, B, total_q), jnp.float32),
        jax.ShapeDtypeStruct((G, B, N), jnp.int32),
        jax.ShapeDtypeStruct((G, B, N, 9), jnp.float32),
    )
    padded_t, labels, bboxes = pl.pallas_call(
        _body,
        out_shape=out_shape,
    )(gt_bboxes_3d, gt_labels, reference_points, rnd, cor)
    padded = jnp.transpose(padded_t, (1, 2, 0))
    rg = jax.lax.broadcasted_iota(jnp.int32, (tgt, 1), 0) // N
    cg = jax.lax.broadcasted_iota(jnp.int32, (1, tgt), 1) // N
    incol = jax.lax.broadcasted_iota(jnp.int32, (1, tgt), 1) < pad
    mask = incol & (rg != cg)
    return (padded, mask,
            labels.reshape(G * B * N), bboxes.reshape(G * B * N, 9))


# speedup vs baseline: 1.2051x; 1.0009x over previous
import jax
import jax.numpy as jnp
from jax.experimental import pallas as pl

_NUM_CLASSES = 10
_NUM_QUERY = 900
_NUM_PROPAGATED = 256
_G = 5
_BBOX_NOISE_SCALE = 0.4
_NOISE_THRESH = 0.5
_PC_LO = (-65.0, -65.0, -8.0)
_PC_RNG = (130.0, 130.0, 16.0)


def _body(gt_ref, lab_ref, rp_ref, rnd_ref,
          padded_t_ref, labels_ref, bboxes_ref):
    B, N = lab_ref.shape
    G = _G
    pad = G * N

    gt = gt_ref[...]
    rp = rp_ref[...]
    for k in range(3):
        c_k = gt[:, :, k]
        d2_k = gt[:, :, 3 + k] * 0.5
        for g in range(G):
            r = rnd_ref[pl.ds((k * G + g) * B, B), :]
            noisy = c_k + r * d2_k * _BBOX_NOISE_SCALE
            noisy = jnp.clip((noisy - _PC_LO[k]) / _PC_RNG[k], 0.0, 1.0)
            padded_t_ref[k, :, g * N:(g + 1) * N] = noisy
        padded_t_ref[k, :, pad:] = jnp.broadcast_to(rp[:, k][None, :],
                                                    (B, _NUM_QUERY))

    lab = lab_ref[...]
    for g in range(G):
        corf = rnd_ref[pl.ds(3 * G * B + g * B, B), :]
        labels_ref[g] = jnp.where(corf != 0.0, _NUM_CLASSES, lab)
        bboxes_ref[g] = gt


def kernel(batch_size, reference_points, gt_bboxes_3d, gt_labels):
    B, N = gt_labels.shape
    G = _G
    pad = G * N
    total_q = pad + _NUM_QUERY
    tgt = total_q + _NUM_PROPAGATED

    with jax.ensure_compile_time_eval():
        nk = jax.random.key(12345)
        rand_prob = jax.random.uniform(nk, (G * B * N, 3), dtype=jnp.float32)
        rand_prob = rand_prob * 2.0 - 1.0
        corrupt = (jnp.linalg.norm(rand_prob, axis=1) > _NOISE_THRESH)
        rnd = jnp.transpose(rand_prob.reshape(G, B, N, 3),
                            (3, 0, 1, 2)).reshape(3 * G * B, N)
        corf = corrupt.astype(jnp.float32).reshape(G * B, N)
        rnd = jnp.concatenate([rnd, corf], axis=0)

    out_shape = (
        jax.ShapeDtypeStruct((3, B, total_q), jnp.float32),
        jax.ShapeDtypeStruct((G, B, N), jnp.int32),
        jax.ShapeDtypeStruct((G, B, N, 9), jnp.float32),
    )
    padded_t, labels, bboxes = pl.pallas_call(
        _body,
        out_shape=out_shape,
    )(gt_bboxes_3d, gt_labels, reference_points, rnd)
    padded = jnp.transpose(padded_t, (1, 2, 0))
    rg = jax.lax.broadcasted_iota(jnp.int32, (tgt, 1), 0) // N
    cg = jax.lax.broadcasted_iota(jnp.int32, (1, tgt), 1) // N
    incol = jax.lax.broadcasted_iota(jnp.int32, (1, tgt), 1) < pad
    mask = incol & (rg != cg)
    return (padded, mask,
            labels.reshape(G * B * N), bboxes.reshape(G * B * N, 9))


# speedup vs baseline: 1.2082x; 1.0026x over previous
import jax
import jax.numpy as jnp
from jax.experimental import pallas as pl

_NUM_CLASSES = 10
_NUM_QUERY = 900
_NUM_PROPAGATED = 256
_G = 5
_BBOX_NOISE_SCALE = 0.4
_NOISE_THRESH = 0.5
_PC_LO = (-65.0, -65.0, -8.0)
_PC_RNG = (130.0, 130.0, 16.0)


def _body(gt_ref, lab_ref, rnd_ref,
          padded_t_ref, labels_ref, bboxes_ref):
    B, N = lab_ref.shape
    G = _G
    pad = G * N

    gt = gt_ref[...]
    for k in range(3):
        c_k = gt[:, :, k]
        d2_k = gt[:, :, 3 + k] * 0.5
        for g in range(G):
            r = rnd_ref[pl.ds((k * G + g) * B, B), :]
            noisy = c_k + r * d2_k * _BBOX_NOISE_SCALE
            noisy = jnp.clip((noisy - _PC_LO[k]) / _PC_RNG[k], 0.0, 1.0)
            padded_t_ref[k, :, g * N:(g + 1) * N] = noisy

    lab = lab_ref[...]
    for g in range(G):
        corf = rnd_ref[pl.ds(3 * G * B + g * B, B), :]
        labels_ref[g] = jnp.where(corf != 0.0, _NUM_CLASSES, lab)
        bboxes_ref[g] = gt


def kernel(batch_size, reference_points, gt_bboxes_3d, gt_labels):
    B, N = gt_labels.shape
    G = _G
    pad = G * N
    total_q = pad + _NUM_QUERY
    tgt = total_q + _NUM_PROPAGATED

    with jax.ensure_compile_time_eval():
        nk = jax.random.key(12345)
        rand_prob = jax.random.uniform(nk, (G * B * N, 3), dtype=jnp.float32)
        rand_prob = rand_prob * 2.0 - 1.0
        corrupt = (jnp.linalg.norm(rand_prob, axis=1) > _NOISE_THRESH)
        rnd = jnp.transpose(rand_prob.reshape(G, B, N, 3),
                            (3, 0, 1, 2)).reshape(3 * G * B, N)
        corf = corrupt.astype(jnp.float32).reshape(G * B, N)
        rnd = jnp.concatenate([rnd, corf], axis=0)

    out_shape = (
        jax.ShapeDtypeStruct((3, B, pad), jnp.float32),
        jax.ShapeDtypeStruct((G, B, N), jnp.int32),
        jax.ShapeDtypeStruct((G, B, N, 9), jnp.float32),
    )
    padded_t, labels, bboxes = pl.pallas_call(
        _body,
        out_shape=out_shape,
    )(gt_bboxes_3d, gt_labels, rnd)
    padded = jnp.concatenate(
        [jnp.transpose(padded_t, (1, 2, 0)),
         jnp.broadcast_to(reference_points[None], (B, _NUM_QUERY, 3))],
        axis=1)
    rg = jax.lax.broadcasted_iota(jnp.int32, (tgt, 1), 0) // N
    cg = jax.lax.broadcasted_iota(jnp.int32, (1, tgt), 1) // N
    incol = jax.lax.broadcasted_iota(jnp.int32, (1, tgt), 1) < pad
    mask = incol & (rg != cg)
    return (padded, mask,
            labels.reshape(G * B * N), bboxes.reshape(G * B * N, 9))


# speedup vs baseline: 1.2630x; 1.0454x over previous
import jax
import jax.numpy as jnp
from jax.experimental import pallas as pl
from jax.experimental.pallas import tpu as pltpu

_NUM_CLASSES = 10
_NUM_QUERY = 900
_NUM_PROPAGATED = 256
_G = 5
_BBOX_NOISE_SCALE = 0.4
_NOISE_THRESH = 0.5
_PC_LO = (-65.0, -65.0, -8.0)
_PC_RNG = (130.0, 130.0, 16.0)


def _body(gt_ref, lab_ref, rp_ref, rnd_ref,
          padded_t_ref, labels_ref, bboxes_ref, sems_ref):
    B, N = lab_ref.shape
    G = _G
    pad = G * N

    copies = []
    for g in range(G):
        cp = pltpu.make_async_copy(gt_ref, bboxes_ref.at[g], sems_ref.at[g])
        cp.start()
        copies.append(cp)

    gt = gt_ref[...]
    rp = rp_ref[...]
    for k in range(3):
        c_k = gt[:, :, k]
        d2_k = gt[:, :, 3 + k] * 0.5
        for g in range(G):
            r = rnd_ref[pl.ds((k * G + g) * B, B), :]
            noisy = c_k + r * d2_k * _BBOX_NOISE_SCALE
            noisy = jnp.clip((noisy - _PC_LO[k]) / _PC_RNG[k], 0.0, 1.0)
            padded_t_ref[k, :, g * N:(g + 1) * N] = noisy
        padded_t_ref[k, :, pad:] = jnp.broadcast_to(rp[:, k][None, :],
                                                    (B, _NUM_QUERY))

    lab = lab_ref[...]
    for g in range(G):
        corf = rnd_ref[pl.ds(3 * G * B + g * B, B), :]
        labels_ref[g] = jnp.where(corf != 0.0, _NUM_CLASSES, lab)
    for cp in copies:
        cp.wait()


def kernel(batch_size, reference_points, gt_bboxes_3d, gt_labels):
    B, N = gt_labels.shape
    G = _G
    pad = G * N
    total_q = pad + _NUM_QUERY
    tgt = total_q + _NUM_PROPAGATED

    with jax.ensure_compile_time_eval():
        nk = jax.random.key(12345)
        rand_prob = jax.random.uniform(nk, (G * B * N, 3), dtype=jnp.float32)
        rand_prob = rand_prob * 2.0 - 1.0
        corrupt = (jnp.linalg.norm(rand_prob, axis=1) > _NOISE_THRESH)
        rnd = jnp.transpose(rand_prob.reshape(G, B, N, 3),
                            (3, 0, 1, 2)).reshape(3 * G * B, N)
        corf = corrupt.astype(jnp.float32).reshape(G * B, N)
        rnd = jnp.concatenate([rnd, corf], axis=0)

    out_shape = (
        jax.ShapeDtypeStruct((3, B, total_q), jnp.float32),
        jax.ShapeDtypeStruct((G, B, N), jnp.int32),
        jax.ShapeDtypeStruct((G, B, N, 9), jnp.float32),
    )
    padded_t, labels, bboxes = pl.pallas_call(
        _body,
        out_shape=out_shape,
        out_specs=(
            pl.BlockSpec((3, B, total_q), lambda: (0, 0, 0)),
            pl.BlockSpec((G, B, N), lambda: (0, 0, 0)),
            pl.BlockSpec(memory_space=pltpu.MemorySpace.HBM),
        ),
        scratch_shapes=[pltpu.SemaphoreType.DMA((G,))],
    )(gt_bboxes_3d, gt_labels, reference_points, rnd)
    padded = jnp.transpose(padded_t, (1, 2, 0))
    rg = jax.lax.broadcasted_iota(jnp.int32, (tgt, 1), 0) // N
    cg = jax.lax.broadcasted_iota(jnp.int32, (1, tgt), 1) // N
    incol = jax.lax.broadcasted_iota(jnp.int32, (1, tgt), 1) < pad
    mask = incol & (rg != cg)
    return (padded, mask,
            labels.reshape(G * B * N), bboxes.reshape(G * B * N, 9))
